# baseline (speedup 1.0000x reference)
import functools

import jax
import jax.numpy as jnp
from jax import lax
from jax.experimental import pallas as pl
from jax.experimental.pallas import tpu as pltpu
from jax.experimental.pallas import tpu_sc as plsc

_INFO = plsc.get_sparse_core_info()
_NC, _NS, _L = _INFO.num_cores, _INFO.num_subcores, _INFO.num_lanes
_NW = _NC * _NS
_CHUNK = 4096


def _make_gather_t(V, D, B):
    dims_per_w = D // _NW
    n_chunks = B // _CHUNK
    mesh = plsc.VectorSubcoreMesh(core_axis_name="c", subcore_axis_name="s")

    @functools.partial(
        pl.kernel,
        mesh=mesh,
        out_type=jax.ShapeDtypeStruct((D, B), jnp.float32),
        scratch_types=[
            pltpu.VMEM((V,), jnp.float32),
            pltpu.VMEM((2, _CHUNK), jnp.int32),
            pltpu.VMEM((2, _CHUNK), jnp.float32),
            pltpu.SemaphoreType.DMA,
            pltpu.SemaphoreType.DMA,
            pltpu.SemaphoreType.DMA,
            pltpu.SemaphoreType.DMA,
        ],
        compiler_params=pltpu.CompilerParams(
            use_tc_tiling_on_sc=True, needs_layout_passes=False
        ),
    )
    def gather_kernel(idx_hbm, table_t_hbm, out_t_hbm, row_v, idx_v, out_v,
                      si0, si1, so0, so1):
        wid = lax.axis_index("s") * _NC + lax.axis_index("c")
        si = (si0, si1)
        so = (so0, so1)
        total = dims_per_w * n_chunks

        def start_idx(cc):
            c = cc % n_chunks
            return pltpu.async_copy(
                idx_hbm.at[pl.ds(c * _CHUNK, _CHUNK)],
                idx_v.at[cc % 2],
                si[cc % 2],
            )

        idx_pend = {0: start_idx(0), 1: start_idx(1)}
        out_pend = {}

        for k in range(dims_per_w):
            d = wid * dims_per_w + k
            pltpu.sync_copy(table_t_hbm.at[d], row_v)
            for c in range(n_chunks):
                cc = k * n_chunks + c
                slot = cc % 2
                idx_pend.pop(cc).wait()
                if cc - 2 in out_pend:
                    out_pend.pop(cc - 2).wait()

                @plsc.parallel_loop(0, _CHUNK // _L, unroll=8)
                def _(j):
                    iv = idx_v[slot, pl.ds(j * _L, _L)]
                    out_v[slot, pl.ds(j * _L, _L)] = plsc.load_gather(
                        row_v, [iv]
                    )

                out_pend[cc] = pltpu.async_copy(
                    out_v.at[slot],
                    out_t_hbm.at[d, pl.ds(c * _CHUNK, _CHUNK)],
                    so[slot],
                )
                if cc + 2 < total:
                    idx_pend[cc + 2] = start_idx(cc + 2)
        for h in out_pend.values():
            h.wait()

    return gather_kernel


def kernel(time_indices, embeddings):
    B = time_indices.shape[0]
    V, D = embeddings.shape
    idx = time_indices.astype(jnp.int32)
    out_t = _make_gather_t(V, D, B)(idx, embeddings.T)
    return out_t.T


# speedup vs baseline: 1.1183x; 1.1183x over previous
import functools

import jax
import jax.numpy as jnp
from jax import lax
from jax.experimental import pallas as pl
from jax.experimental.pallas import tpu as pltpu
from jax.experimental.pallas import tpu_sc as plsc

_INFO = plsc.get_sparse_core_info()
_NC, _NS, _L = _INFO.num_cores, _INFO.num_subcores, _INFO.num_lanes
_NW = _NC * _NS
_CHUNK = 4096


def _make_gather_t(V, D, B):
    dims_per_w = D // _NW
    n_chunks = B // _CHUNK
    mesh = plsc.VectorSubcoreMesh(core_axis_name="c", subcore_axis_name="s")

    @functools.partial(
        pl.kernel,
        mesh=mesh,
        out_type=jax.ShapeDtypeStruct((D, B), jnp.float32),
        scratch_types=[
            pltpu.VMEM((V,), jnp.float32),
            pltpu.VMEM((B,), jnp.int32),
            pltpu.VMEM((2, _CHUNK), jnp.float32),
            pltpu.SemaphoreType.DMA,
            pltpu.SemaphoreType.DMA,
        ],
        compiler_params=pltpu.CompilerParams(
            use_tc_tiling_on_sc=True, needs_layout_passes=False
        ),
    )
    def gather_kernel(idx_hbm, table_t_hbm, out_t_hbm, row_v, idx_v, out_v,
                      so0, so1):
        wid = lax.axis_index("s") * _NC + lax.axis_index("c")
        so = (so0, so1)
        pend = {}
        pltpu.sync_copy(idx_hbm, idx_v)
        for k in range(dims_per_w):
            d = wid * dims_per_w + k
            pltpu.sync_copy(table_t_hbm.at[d], row_v)
            for c in range(n_chunks):
                cc = k * n_chunks + c
                slot = cc % 2
                if cc - 2 in pend:
                    pend.pop(cc - 2).wait()

                @plsc.parallel_loop(0, _CHUNK // _L, unroll=8)
                def _(j):
                    iv = idx_v[pl.ds(c * _CHUNK + j * _L, _L)]
                    out_v[slot, pl.ds(j * _L, _L)] = plsc.load_gather(row_v, [iv])

                pend[cc] = pltpu.async_copy(
                    out_v.at[slot],
                    out_t_hbm.at[d, pl.ds(c * _CHUNK, _CHUNK)],
                    so[slot],
                )
        for h in pend.values():
            h.wait()

    return gather_kernel


def kernel(time_indices, embeddings):
    B = time_indices.shape[0]
    V, D = embeddings.shape
    idx = time_indices.astype(jnp.int32)
    out_t = _make_gather_t(V, D, B)(idx, embeddings.T)
    return out_t.T


# speedup vs baseline: 1.1788x; 1.0541x over previous
import functools

import jax
import jax.numpy as jnp
from jax import lax
from jax.experimental import pallas as pl
from jax.experimental.pallas import tpu as pltpu
from jax.experimental.pallas import tpu_sc as plsc

_INFO = plsc.get_sparse_core_info()
_NC, _NS, _L = _INFO.num_cores, _INFO.num_subcores, _INFO.num_lanes
_NW = _NC * _NS
_CHUNK = 4096


def _make_gather_t(V, D, B):
    dims_per_w = D // _NW
    n_chunks = B // _CHUNK
    mesh = plsc.VectorSubcoreMesh(core_axis_name="c", subcore_axis_name="s")

    @functools.partial(
        pl.kernel,
        mesh=mesh,
        out_type=jax.ShapeDtypeStruct((D, B), jnp.float32),
        scratch_types=[
            pltpu.VMEM((V,), jnp.float32),
            pltpu.VMEM((B,), jnp.int32),
            pltpu.VMEM((2, _CHUNK), jnp.float32),
            pltpu.VMEM_SHARED((B,), jnp.int32),
            pltpu.SemaphoreType.DMA,
            pltpu.SemaphoreType.DMA,
        ],
        compiler_params=pltpu.CompilerParams(
            use_tc_tiling_on_sc=True, needs_layout_passes=False
        ),
    )
    def gather_kernel(idx_hbm, table_t_hbm, out_t_hbm, row_v, idx_v, out_v,
                      idx_s, so0, so1):
        wid = lax.axis_index("s") * _NC + lax.axis_index("c")
        so = (so0, so1)
        pend = {}

        @pl.when(lax.axis_index("s") == 0)
        def _():
            pltpu.sync_copy(idx_hbm, idx_s)

        plsc.subcore_barrier()
        pltpu.sync_copy(idx_s, idx_v)
        for k in range(dims_per_w):
            d = wid * dims_per_w + k
            pltpu.sync_copy(table_t_hbm.at[d], row_v)
            for c in range(n_chunks):
                cc = k * n_chunks + c
                slot = cc % 2
                if cc - 2 in pend:
                    pend.pop(cc - 2).wait()

                @plsc.parallel_loop(0, _CHUNK // _L, unroll=8)
                def _(j):
                    iv = idx_v[pl.ds(c * _CHUNK + j * _L, _L)]
                    out_v[slot, pl.ds(j * _L, _L)] = plsc.load_gather(row_v, [iv])

                pend[cc] = pltpu.async_copy(
                    out_v.at[slot],
                    out_t_hbm.at[d, pl.ds(c * _CHUNK, _CHUNK)],
                    so[slot],
                )
        for h in pend.values():
            h.wait()

    return gather_kernel


def kernel(time_indices, embeddings):
    B = time_indices.shape[0]
    V, D = embeddings.shape
    idx = time_indices.astype(jnp.int32)
    out_t = _make_gather_t(V, D, B)(idx, embeddings.T)
    return out_t.T


# speedup vs baseline: 1.1809x; 1.0018x over previous
import functools

import jax
import jax.numpy as jnp
from jax import lax
from jax.experimental import pallas as pl
from jax.experimental.pallas import tpu as pltpu
from jax.experimental.pallas import tpu_sc as plsc

_INFO = plsc.get_sparse_core_info()
_NC, _NS, _L = _INFO.num_cores, _INFO.num_subcores, _INFO.num_lanes
_NW = _NC * _NS
_CHUNK = 4096


def _make_gather_t(V, D, B):
    dims_per_w = D // _NW
    n_chunks = B // _CHUNK
    mesh = plsc.VectorSubcoreMesh(core_axis_name="c", subcore_axis_name="s")

    @functools.partial(
        pl.kernel,
        mesh=mesh,
        out_type=jax.ShapeDtypeStruct((D, B), jnp.float32),
        scratch_types=[
            pltpu.VMEM((V,), jnp.float32),
            pltpu.VMEM((B,), jnp.int32),
            pltpu.VMEM((2, _CHUNK), jnp.float32),
            pltpu.VMEM_SHARED((B,), jnp.int32),
            pltpu.SemaphoreType.DMA,
            pltpu.SemaphoreType.DMA,
        ],
        compiler_params=pltpu.CompilerParams(
            use_tc_tiling_on_sc=True, needs_layout_passes=False
        ),
    )
    def gather_kernel(idx_hbm, table_t_hbm, out_t_hbm, row_v, idx_v, out_v,
                      idx_s, so0, so1):
        wid = lax.axis_index("s") * _NC + lax.axis_index("c")
        so = (so0, so1)
        pend = {}

        with jax.named_scope("idx_stage"):
            @pl.when(lax.axis_index("s") == 0)
            def _():
                pltpu.sync_copy(idx_hbm, idx_s)

            plsc.subcore_barrier()
            pltpu.sync_copy(idx_s, idx_v)
        for k in range(dims_per_w):
            d = wid * dims_per_w + k
            with jax.named_scope("row_dma"):
                pltpu.sync_copy(table_t_hbm.at[d], row_v)
            for c in range(n_chunks):
                cc = k * n_chunks + c
                slot = cc % 2
                if cc - 2 in pend:
                    pend.pop(cc - 2).wait()

                with jax.named_scope("gather"):
                    @plsc.parallel_loop(0, _CHUNK // _L, unroll=8)
                    def _(j):
                        iv = idx_v[pl.ds(c * _CHUNK + j * _L, _L)]
                        out_v[slot, pl.ds(j * _L, _L)] = plsc.load_gather(row_v, [iv])

                pend[cc] = pltpu.async_copy(
                    out_v.at[slot],
                    out_t_hbm.at[d, pl.ds(c * _CHUNK, _CHUNK)],
                    so[slot],
                )
        with jax.named_scope("drain"):
            for h in pend.values():
                h.wait()

    return gather_kernel


def kernel(time_indices, embeddings):
    B = time_indices.shape[0]
    V, D = embeddings.shape
    idx = time_indices.astype(jnp.int32)
    out_t = _make_gather_t(V, D, B)(idx, embeddings.T)
    return out_t.T


# speedup vs baseline: 1.2363x; 1.0470x over previous
import functools

import jax
import jax.numpy as jnp
from jax import lax
from jax.experimental import pallas as pl
from jax.experimental.pallas import tpu as pltpu
from jax.experimental.pallas import tpu_sc as plsc

_INFO = plsc.get_sparse_core_info()
_NC, _NS, _L = _INFO.num_cores, _INFO.num_subcores, _INFO.num_lanes
_NW = _NC * _NS
_CHUNK = 4096


def _make_gather_t(V, D, B):
    dims_per_w = D // _NW
    n_chunks = B // _CHUNK
    mesh = plsc.VectorSubcoreMesh(core_axis_name="c", subcore_axis_name="s")

    @functools.partial(
        pl.kernel,
        mesh=mesh,
        out_type=jax.ShapeDtypeStruct((D, B), jnp.float32),
        scratch_types=[
            pltpu.VMEM((V,), jnp.float32),
            pltpu.VMEM((B,), jnp.int32),
            pltpu.VMEM((2, _CHUNK), jnp.float32),
            pltpu.VMEM_SHARED((B,), jnp.int32),
            pltpu.SemaphoreType.DMA,
            pltpu.SemaphoreType.DMA,
            pltpu.SemaphoreType.DMA,
        ],
        compiler_params=pltpu.CompilerParams(
            use_tc_tiling_on_sc=True, needs_layout_passes=False
        ),
    )
    def gather_kernel(idx_hbm, table_t_hbm, out_t_hbm, row_v, idx_v, out_v,
                      idx_s, so0, so1, sr):
        wid = lax.axis_index("s") * _NC + lax.axis_index("c")
        so = (so0, so1)
        pend = {}
        row_h = pltpu.async_copy(
            table_t_hbm.at[wid * dims_per_w], row_v, sr
        )

        @pl.when(lax.axis_index("s") == 0)
        def _():
            pltpu.sync_copy(idx_hbm, idx_s)

        plsc.subcore_barrier()
        pltpu.sync_copy(idx_s, idx_v)
        for k in range(dims_per_w):
            d = wid * dims_per_w + k
            if k == 0:
                row_h.wait()
            else:
                pltpu.sync_copy(table_t_hbm.at[d], row_v)
            for c in range(n_chunks):
                cc = k * n_chunks + c
                slot = cc % 2
                if cc - 2 in pend:
                    pend.pop(cc - 2).wait()

                @plsc.parallel_loop(0, _CHUNK // _L, unroll=8)
                def _(j):
                    iv = idx_v[pl.ds(c * _CHUNK + j * _L, _L)]
                    out_v[slot, pl.ds(j * _L, _L)] = plsc.load_gather(row_v, [iv])

                pend[cc] = pltpu.async_copy(
                    out_v.at[slot],
                    out_t_hbm.at[d, pl.ds(c * _CHUNK, _CHUNK)],
                    so[slot],
                )
        for h in pend.values():
            h.wait()

    return gather_kernel


def kernel(time_indices, embeddings):
    B = time_indices.shape[0]
    V, D = embeddings.shape
    idx = time_indices.astype(jnp.int32)
    out_t = _make_gather_t(V, D, B)(idx, embeddings.T)
    return out_t.T
